# baseline (speedup 1.0000x reference)
import math

import jax
import jax.numpy as jnp
from jax.experimental import pallas as pl

_NUM_CLASSES = 80
_IOU_WEIGHT = 3.0
_LAMBDA_BOX = 7.5
_LAMBDA_CLS = 0.5
_I32_MIN = -(2**31)
_I32_MAX = 2**31 - 1


def _sortable_key(cost):
    bits = jax.lax.bitcast_convert_type(cost, jnp.int32)
    return bits ^ (jnp.right_shift(bits, 31) & 0x7FFFFFFF)


def _arctan(x):
    s = jnp.sign(x)
    ax = jnp.abs(x)
    inv = ax > 1.0
    t = jnp.where(inv, 1.0 / jnp.maximum(ax, 1e-30), ax)
    t2 = t * t
    p = -0.0040540580
    for coef in (0.0218612288, -0.0559098861, 0.0964200441, -0.1390853351,
                 0.1994653599, -0.3332985605, 0.9999993329):
        p = p * t2 + coef
    r = t * p
    return s * jnp.where(inv, math.pi / 2 - r, r)


def _softplus(x):
    return jnp.maximum(x, 0.0) + jnp.log1p(jnp.exp(-jnp.abs(x)))


def _rowcount(mask):
    a = mask.shape[1]
    ones = jnp.ones((a, 1), jnp.bfloat16)
    return jax.lax.dot_general(
        mask.astype(jnp.bfloat16), ones, (((1,), (0,)), ((), ())),
        preferred_element_type=jnp.float32)


def _fused(scores_ref, pdbT_ref, ancT_ref, gtb_ref, cls_ref, loss_ref):
    g = gtb_ref.shape[0]
    a, c = scores_ref.shape

    gx1 = gtb_ref[:, 0:1]
    gy1 = gtb_ref[:, 1:2]
    gx2 = gtb_ref[:, 2:3]
    gy2 = gtb_ref[:, 3:4]
    px1 = pdbT_ref[0:1, :]
    py1 = pdbT_ref[1:2, :]
    px2 = pdbT_ref[2:3, :]
    py2 = pdbT_ref[3:4, :]
    ax = ancT_ref[0:1, :]
    ay = ancT_ref[1:2, :]
    ix1 = jnp.maximum(gx1, px1)
    iy1 = jnp.maximum(gy1, py1)
    ix2 = jnp.minimum(gx2, px2)
    iy2 = jnp.minimum(gy2, py2)
    inter = jnp.clip(ix2 - ix1, 0, None) * jnp.clip(iy2 - iy1, 0, None)
    a1 = (gx2 - gx1) * (gy2 - gy1)
    a2 = (px2 - px1) * (py2 - py1)
    iou = inter / (a1 + a2 - inter + 1e-9)
    is_in = jnp.minimum(jnp.minimum(ax - gx1, ay - gy1),
                        jnp.minimum(gx2 - ax, gy2 - ay)) > 0.01
    iota_c = jax.lax.broadcasted_iota(jnp.int32, (1, c), 1)
    onehot = (cls_ref[...] == iota_c).astype(jnp.float32)
    pps = jax.lax.dot_general(
        onehot, scores_ref[...], (((1,), (1,)), ((), ())),
        precision=jax.lax.Precision.HIGHEST,
        preferred_element_type=jnp.float32)
    cost = (_softplus(-pps) - _IOU_WEIGHT * jnp.log(iou + 1e-8)
            + 1e5 * (~is_in).astype(jnp.float32))
    key = _sortable_key(cost)
    iou_m = jnp.where(is_in, iou, 0.0)

    k = jnp.clip(iou_m.sum(axis=1, keepdims=True).astype(jnp.int32), 1, a)
    kf = k.astype(jnp.float32)

    def body_val(_, lh):
        lo, hi = lh
        mid = lo + jax.lax.shift_right_logical(hi - lo, 1)
        pred = _rowcount(key <= mid) >= kf
        return (jnp.where(pred, lo, mid + 1), jnp.where(pred, mid, hi))

    t, _ = jax.lax.fori_loop(
        0, 32, body_val,
        (jnp.full((g, 1), _I32_MIN, jnp.int32),
         jnp.full((g, 1), _I32_MAX, jnp.int32)))
    n_lt = _rowcount(key < t)
    need = kf - n_lt
    n_eq = _rowcount(key == t)

    def body_idx(_, lh):
        lo, hi = lh
        mid = lo + jax.lax.shift_right_logical(hi - lo, 1)
        idx_b = jax.lax.broadcasted_iota(jnp.int32, (g, a), 1)
        pred = _rowcount((key == t) & (idx_b <= mid)) >= need
        return (jnp.where(pred, lo, mid + 1), jnp.where(pred, mid, hi))

    def _tie_search(_):
        lo2, _ = jax.lax.fori_loop(
            0, 15, body_idx,
            (jnp.zeros((g, 1), jnp.int32),
             jnp.full((g, 1), a - 1, jnp.int32)))
        return lo2

    tie_i = jax.lax.cond(jnp.all(n_eq == need), lambda _: jnp.full(
        (g, 1), a - 1, jnp.int32), _tie_search, operand=0)

    idx = jax.lax.broadcasted_iota(jnp.int32, (g, a), 1)
    mm = (key < t) | ((key == t) & (idx <= tie_i))
    mmf = mm.astype(jnp.float32)
    amg = mmf.sum(axis=0, keepdims=True)
    conflict_f = (amg > 1.0).astype(jnp.float32)
    minkey = key.min(axis=0, keepdims=True)
    iota_g = jax.lax.broadcasted_iota(jnp.int32, (g, a), 0)
    cm = jnp.where(key == minkey, iota_g, jnp.int32(g)).min(
        axis=0, keepdims=True)
    onehot_f = (iota_g == cm).astype(jnp.float32)
    mmf2 = conflict_f * onehot_f + (1.0 - conflict_f) * mmf

    matched_iou = (mmf2 * iou_m).sum(axis=0, keepdims=True)
    dot_s = (mmf2 * iou_m * pps).sum()
    sp_s = _softplus(scores_ref[...]).sum()
    s1 = matched_iou.sum()

    bx1 = (mmf2 * gx1).sum(axis=0, keepdims=True)
    by1 = (mmf2 * gy1).sum(axis=0, keepdims=True)
    bx2 = (mmf2 * gx2).sum(axis=0, keepdims=True)
    by2 = (mmf2 * gy2).sum(axis=0, keepdims=True)
    eps = 1e-7
    w1 = px2 - px1
    h1 = py2 - py1 + eps
    w2 = bx2 - bx1
    h2 = by2 - by1 + eps
    inter2 = (jnp.clip(jnp.minimum(px2, bx2) - jnp.maximum(px1, bx1), 0, None)
              * jnp.clip(jnp.minimum(py2, by2) - jnp.maximum(py1, by1), 0, None))
    union2 = w1 * h1 + w2 * h2 - inter2 + eps
    iou2 = inter2 / union2
    cw = jnp.maximum(px2, bx2) - jnp.minimum(px1, bx1)
    ch = jnp.maximum(py2, by2) - jnp.minimum(py1, by1)
    c2 = cw * cw + ch * ch + eps
    rho2 = ((bx1 + bx2 - px1 - px2) ** 2 + (by1 + by2 - py1 - py2) ** 2) / 4.0
    v = (4.0 / math.pi**2) * (_arctan(w2 / h2) - _arctan(w1 / h1)) ** 2
    alpha = v / (v - iou2 + (1.0 + eps))
    ciou = iou2 - (rho2 / c2 + v * alpha)
    box_s = ((1.0 - ciou) * matched_iou).sum()

    tss = jnp.maximum(s1, 1.0)
    loss = (_LAMBDA_BOX * box_s + _LAMBDA_CLS * (sp_s - dot_s)) / tss
    loss_ref[...] = loss.reshape(1, 1)


def kernel(pd_scores, pd_bboxes, anc_points, gt_labels, gt_bboxes):
    cls2d = gt_labels[:, 0:1].astype(jnp.int32)
    out = pl.pallas_call(
        _fused,
        out_shape=jax.ShapeDtypeStruct((1, 1), jnp.float32),
    )(pd_scores, pd_bboxes.T, anc_points.T, gt_bboxes, cls2d)
    return out[0, 0]


# speedup vs baseline: 1.1780x; 1.1780x over previous
import math

import jax
import jax.numpy as jnp
from jax.experimental import pallas as pl

_NUM_CLASSES = 80
_IOU_WEIGHT = 3.0
_LAMBDA_BOX = 7.5
_LAMBDA_CLS = 0.5
_I32_MIN = -(2**31)
_I32_MAX = 2**31 - 1


def _sortable_key(cost):
    bits = jax.lax.bitcast_convert_type(cost, jnp.int32)
    return bits ^ (jnp.right_shift(bits, 31) & 0x7FFFFFFF)


def _arctan(x):
    s = jnp.sign(x)
    ax = jnp.abs(x)
    inv = ax > 1.0
    t = jnp.where(inv, 1.0 / jnp.maximum(ax, 1e-30), ax)
    t2 = t * t
    p = -0.0040540580
    for coef in (0.0218612288, -0.0559098861, 0.0964200441, -0.1390853351,
                 0.1994653599, -0.3332985605, 0.9999993329):
        p = p * t2 + coef
    r = t * p
    return s * jnp.where(inv, math.pi / 2 - r, r)


def _softplus(x):
    return jnp.maximum(x, 0.0) + jnp.log1p(jnp.exp(-jnp.abs(x)))


def _rowcount(mask):
    g, a = mask.shape
    step = 2560
    parts = [mask[:, o:min(o + step, a)].astype(jnp.int32).sum(
        axis=1, keepdims=True) for o in range(0, a, step)]
    while len(parts) > 1:
        parts = [parts[i] + parts[i + 1] if i + 1 < len(parts) else parts[i]
                 for i in range(0, len(parts), 2)]
    return parts[0]


def _fused(scores_ref, pdbT_ref, ancT_ref, gtb_ref, cls_ref, loss_ref):
    g = gtb_ref.shape[0]
    a, c = scores_ref.shape

    gx1 = gtb_ref[:, 0:1]
    gy1 = gtb_ref[:, 1:2]
    gx2 = gtb_ref[:, 2:3]
    gy2 = gtb_ref[:, 3:4]
    px1 = pdbT_ref[0:1, :]
    py1 = pdbT_ref[1:2, :]
    px2 = pdbT_ref[2:3, :]
    py2 = pdbT_ref[3:4, :]
    ax = ancT_ref[0:1, :]
    ay = ancT_ref[1:2, :]
    ix1 = jnp.maximum(gx1, px1)
    iy1 = jnp.maximum(gy1, py1)
    ix2 = jnp.minimum(gx2, px2)
    iy2 = jnp.minimum(gy2, py2)
    inter = jnp.clip(ix2 - ix1, 0, None) * jnp.clip(iy2 - iy1, 0, None)
    a1 = (gx2 - gx1) * (gy2 - gy1)
    a2 = (px2 - px1) * (py2 - py1)
    iou = inter / (a1 + a2 - inter + 1e-9)
    is_in = jnp.minimum(jnp.minimum(ax - gx1, ay - gy1),
                        jnp.minimum(gx2 - ax, gy2 - ay)) > 0.01
    iota_c = jax.lax.broadcasted_iota(jnp.int32, (1, c), 1)
    onehot = (cls_ref[...] == iota_c).astype(jnp.float32)
    pps = jax.lax.dot_general(
        onehot, scores_ref[...], (((1,), (1,)), ((), ())),
        precision=jax.lax.Precision.HIGHEST,
        preferred_element_type=jnp.float32)
    cost = (_softplus(-pps) - _IOU_WEIGHT * jnp.log(iou + 1e-8)
            + 1e5 * (~is_in).astype(jnp.float32))
    key = _sortable_key(cost)
    iou_m = jnp.where(is_in, iou, 0.0)

    k = jnp.clip(iou_m.sum(axis=1, keepdims=True).astype(jnp.int32), 1, a)
    def body_val(_, lh):
        lo, hi = lh
        mid = lo + jax.lax.shift_right_logical(hi - lo, 1)
        pred = _rowcount(key <= mid) >= k
        return (jnp.where(pred, lo, mid + 1), jnp.where(pred, mid, hi))

    t, _ = jax.lax.fori_loop(
        0, 32, body_val,
        (jnp.full((g, 1), _I32_MIN, jnp.int32),
         jnp.full((g, 1), _I32_MAX, jnp.int32)))
    n_lt = _rowcount(key < t)
    need = k - n_lt
    n_eq = _rowcount(key == t)

    def body_idx(_, lh):
        lo, hi = lh
        mid = lo + jax.lax.shift_right_logical(hi - lo, 1)
        idx_b = jax.lax.broadcasted_iota(jnp.int32, (g, a), 1)
        pred = _rowcount((key == t) & (idx_b <= mid)) >= need
        return (jnp.where(pred, lo, mid + 1), jnp.where(pred, mid, hi))

    def _tie_search(_):
        lo2, _ = jax.lax.fori_loop(
            0, 15, body_idx,
            (jnp.zeros((g, 1), jnp.int32),
             jnp.full((g, 1), a - 1, jnp.int32)))
        return lo2

    tie_i = jax.lax.cond(jnp.all(n_eq == need), lambda _: jnp.full(
        (g, 1), a - 1, jnp.int32), _tie_search, operand=0)

    idx = jax.lax.broadcasted_iota(jnp.int32, (g, a), 1)
    mm = (key < t) | ((key == t) & (idx <= tie_i))
    mmf = mm.astype(jnp.float32)
    ones_g = jnp.ones((1, g), jnp.bfloat16)
    amg = jax.lax.dot_general(
        ones_g, mm.astype(jnp.bfloat16), (((1,), (0,)), ((), ())),
        preferred_element_type=jnp.float32)
    conflict_f = (amg > 1.0).astype(jnp.float32)
    minkey = key.min(axis=0, keepdims=True)
    iota_g = jax.lax.broadcasted_iota(jnp.int32, (g, a), 0)
    cm = jnp.where(key == minkey, iota_g, jnp.int32(g)).min(
        axis=0, keepdims=True)
    onehot_f = (iota_g == cm).astype(jnp.float32)
    mmf2 = conflict_f * onehot_f + (1.0 - conflict_f) * mmf

    matched_iou = (mmf2 * iou_m).sum(axis=0, keepdims=True)
    dot_s = (mmf2 * iou_m * pps).sum()
    sp_s = _softplus(scores_ref[...]).sum()
    s1 = matched_iou.sum()

    gtbT = jnp.transpose(gtb_ref[...], (1, 0))
    bxy = jax.lax.dot_general(
        gtbT, mmf2, (((1,), (0,)), ((), ())),
        precision=jax.lax.Precision.HIGHEST,
        preferred_element_type=jnp.float32)
    bx1 = bxy[0:1, :]
    by1 = bxy[1:2, :]
    bx2 = bxy[2:3, :]
    by2 = bxy[3:4, :]
    eps = 1e-7
    w1 = px2 - px1
    h1 = py2 - py1 + eps
    w2 = bx2 - bx1
    h2 = by2 - by1 + eps
    inter2 = (jnp.clip(jnp.minimum(px2, bx2) - jnp.maximum(px1, bx1), 0, None)
              * jnp.clip(jnp.minimum(py2, by2) - jnp.maximum(py1, by1), 0, None))
    union2 = w1 * h1 + w2 * h2 - inter2 + eps
    iou2 = inter2 / union2
    cw = jnp.maximum(px2, bx2) - jnp.minimum(px1, bx1)
    ch = jnp.maximum(py2, by2) - jnp.minimum(py1, by1)
    c2 = cw * cw + ch * ch + eps
    rho2 = ((bx1 + bx2 - px1 - px2) ** 2 + (by1 + by2 - py1 - py2) ** 2) / 4.0
    v = (4.0 / math.pi**2) * (_arctan(w2 / h2) - _arctan(w1 / h1)) ** 2
    alpha = v / (v - iou2 + (1.0 + eps))
    ciou = iou2 - (rho2 / c2 + v * alpha)
    box_s = ((1.0 - ciou) * matched_iou).sum()

    tss = jnp.maximum(s1, 1.0)
    loss = (_LAMBDA_BOX * box_s + _LAMBDA_CLS * (sp_s - dot_s)) / tss
    loss_ref[...] = loss.reshape(1, 1)


def kernel(pd_scores, pd_bboxes, anc_points, gt_labels, gt_bboxes):
    cls2d = gt_labels[:, 0:1].astype(jnp.int32)
    out = pl.pallas_call(
        _fused,
        out_shape=jax.ShapeDtypeStruct((1, 1), jnp.float32),
    )(pd_scores, pd_bboxes.T, anc_points.T, gt_bboxes, cls2d)
    return out[0, 0]


# speedup vs baseline: 1.4121x; 1.1987x over previous
import math

import jax
import jax.numpy as jnp
from jax.experimental import pallas as pl

_NUM_CLASSES = 80
_IOU_WEIGHT = 3.0
_LAMBDA_BOX = 7.5
_LAMBDA_CLS = 0.5
_I32_MIN = -(2**31)
_I32_MAX = 2**31 - 1


def _sortable_key(cost):
    bits = jax.lax.bitcast_convert_type(cost, jnp.int32)
    return bits ^ (jnp.right_shift(bits, 31) & 0x7FFFFFFF)


def _arctan(x):
    s = jnp.sign(x)
    ax = jnp.abs(x)
    inv = ax > 1.0
    t = jnp.where(inv, 1.0 / jnp.maximum(ax, 1e-30), ax)
    t2 = t * t
    p = -0.0040540580
    for coef in (0.0218612288, -0.0559098861, 0.0964200441, -0.1390853351,
                 0.1994653599, -0.3332985605, 0.9999993329):
        p = p * t2 + coef
    r = t * p
    return s * jnp.where(inv, math.pi / 2 - r, r)


def _softplus(x):
    return jnp.maximum(x, 0.0) + jnp.log1p(jnp.exp(-jnp.abs(x)))


def _rowcount(mask):
    g, a = mask.shape
    step = 2560
    parts = [mask[:, o:min(o + step, a)].astype(jnp.int32).sum(
        axis=1, keepdims=True) for o in range(0, a, step)]
    while len(parts) > 1:
        parts = [parts[i] + parts[i + 1] if i + 1 < len(parts) else parts[i]
                 for i in range(0, len(parts), 2)]
    return parts[0]


def _fused(scores_ref, pdbT_ref, ancT_ref, gtb_ref, cls_ref, loss_ref):
    g = gtb_ref.shape[0]
    a, c = scores_ref.shape

    gx1 = gtb_ref[:, 0:1]
    gy1 = gtb_ref[:, 1:2]
    gx2 = gtb_ref[:, 2:3]
    gy2 = gtb_ref[:, 3:4]
    px1 = pdbT_ref[0:1, :]
    py1 = pdbT_ref[1:2, :]
    px2 = pdbT_ref[2:3, :]
    py2 = pdbT_ref[3:4, :]
    ax = ancT_ref[0:1, :]
    ay = ancT_ref[1:2, :]
    ix1 = jnp.maximum(gx1, px1)
    iy1 = jnp.maximum(gy1, py1)
    ix2 = jnp.minimum(gx2, px2)
    iy2 = jnp.minimum(gy2, py2)
    inter = jnp.clip(ix2 - ix1, 0, None) * jnp.clip(iy2 - iy1, 0, None)
    a1 = (gx2 - gx1) * (gy2 - gy1)
    a2 = (px2 - px1) * (py2 - py1)
    iou = inter / (a1 + a2 - inter + 1e-9)
    is_in = jnp.minimum(jnp.minimum(ax - gx1, ay - gy1),
                        jnp.minimum(gx2 - ax, gy2 - ay)) > 0.01
    iota_c = jax.lax.broadcasted_iota(jnp.int32, (1, c), 1)
    onehot = (cls_ref[...] == iota_c).astype(jnp.float32)
    pps = jax.lax.dot_general(
        onehot, scores_ref[...], (((1,), (1,)), ((), ())),
        precision=jax.lax.Precision.HIGHEST,
        preferred_element_type=jnp.float32)
    cost = (_softplus(-pps) - _IOU_WEIGHT * jnp.log(iou + 1e-8)
            + 1e5 * (~is_in).astype(jnp.float32))
    key = _sortable_key(cost)
    iou_m = jnp.where(is_in, iou, 0.0)

    k = jnp.clip(iou_m.sum(axis=1, keepdims=True).astype(jnp.int32), 1, a)
    def cond_val(lh):
        lo, hi = lh
        return jnp.any(lo < hi)

    def body_val(lh):
        lo, hi = lh
        mid = lo + jax.lax.shift_right_logical(hi - lo, 1)
        pred = _rowcount(key <= mid) >= k
        return (jnp.where(pred, lo, mid + 1), jnp.where(pred, mid, hi))

    t, _ = jax.lax.while_loop(
        cond_val, body_val,
        (key.min(axis=1, keepdims=True), key.max(axis=1, keepdims=True)))
    n_lt = _rowcount(key < t)
    need = k - n_lt
    n_eq = _rowcount(key == t)

    def body_idx(_, lh):
        lo, hi = lh
        mid = lo + jax.lax.shift_right_logical(hi - lo, 1)
        idx_b = jax.lax.broadcasted_iota(jnp.int32, (g, a), 1)
        pred = _rowcount((key == t) & (idx_b <= mid)) >= need
        return (jnp.where(pred, lo, mid + 1), jnp.where(pred, mid, hi))

    def _tie_search(_):
        lo2, _ = jax.lax.fori_loop(
            0, 15, body_idx,
            (jnp.zeros((g, 1), jnp.int32),
             jnp.full((g, 1), a - 1, jnp.int32)))
        return lo2

    tie_i = jax.lax.cond(jnp.all(n_eq == need), lambda _: jnp.full(
        (g, 1), a - 1, jnp.int32), _tie_search, operand=0)

    idx = jax.lax.broadcasted_iota(jnp.int32, (g, a), 1)
    mm = (key < t) | ((key == t) & (idx <= tie_i))
    mmf = mm.astype(jnp.float32)
    ones_g = jnp.ones((1, g), jnp.bfloat16)
    amg = jax.lax.dot_general(
        ones_g, mm.astype(jnp.bfloat16), (((1,), (0,)), ((), ())),
        preferred_element_type=jnp.float32)
    conflict_f = (amg > 1.0).astype(jnp.float32)
    minkey = key.min(axis=0, keepdims=True)
    iota_g = jax.lax.broadcasted_iota(jnp.int32, (g, a), 0)
    cm = jnp.where(key == minkey, iota_g, jnp.int32(g)).min(
        axis=0, keepdims=True)
    onehot_f = (iota_g == cm).astype(jnp.float32)
    mmf2 = conflict_f * onehot_f + (1.0 - conflict_f) * mmf

    matched_iou = (mmf2 * iou_m).sum(axis=0, keepdims=True)
    dot_s = (mmf2 * iou_m * pps).sum()
    sp_s = _softplus(scores_ref[...]).sum()
    s1 = matched_iou.sum()

    gtbT = jnp.transpose(gtb_ref[...], (1, 0))
    bxy = jax.lax.dot_general(
        gtbT, mmf2, (((1,), (0,)), ((), ())),
        precision=jax.lax.Precision.HIGHEST,
        preferred_element_type=jnp.float32)
    bx1 = bxy[0:1, :]
    by1 = bxy[1:2, :]
    bx2 = bxy[2:3, :]
    by2 = bxy[3:4, :]
    eps = 1e-7
    w1 = px2 - px1
    h1 = py2 - py1 + eps
    w2 = bx2 - bx1
    h2 = by2 - by1 + eps
    inter2 = (jnp.clip(jnp.minimum(px2, bx2) - jnp.maximum(px1, bx1), 0, None)
              * jnp.clip(jnp.minimum(py2, by2) - jnp.maximum(py1, by1), 0, None))
    union2 = w1 * h1 + w2 * h2 - inter2 + eps
    iou2 = inter2 / union2
    cw = jnp.maximum(px2, bx2) - jnp.minimum(px1, bx1)
    ch = jnp.maximum(py2, by2) - jnp.minimum(py1, by1)
    c2 = cw * cw + ch * ch + eps
    rho2 = ((bx1 + bx2 - px1 - px2) ** 2 + (by1 + by2 - py1 - py2) ** 2) / 4.0
    v = (4.0 / math.pi**2) * (_arctan(w2 / h2) - _arctan(w1 / h1)) ** 2
    alpha = v / (v - iou2 + (1.0 + eps))
    ciou = iou2 - (rho2 / c2 + v * alpha)
    box_s = ((1.0 - ciou) * matched_iou).sum()

    tss = jnp.maximum(s1, 1.0)
    loss = (_LAMBDA_BOX * box_s + _LAMBDA_CLS * (sp_s - dot_s)) / tss
    loss_ref[...] = loss.reshape(1, 1)


def kernel(pd_scores, pd_bboxes, anc_points, gt_labels, gt_bboxes):
    cls2d = gt_labels[:, 0:1].astype(jnp.int32)
    out = pl.pallas_call(
        _fused,
        out_shape=jax.ShapeDtypeStruct((1, 1), jnp.float32),
    )(pd_scores, pd_bboxes.T, anc_points.T, gt_bboxes, cls2d)
    return out[0, 0]
